# Initial kernel scaffold; baseline (speedup 1.0000x reference)
import functools

import jax
import jax.numpy as jnp
from jax.experimental import pallas as pl

NOBJ = 40
NSEG = 48
NBLK = 512
NEG = -3.0e38


def _tile_kernel(params_ref, pts_ref, feat_ref, pf_ref, seg_ref, cnt_ref,
                 *, nblocks):
    k = pl.program_id(1)

    feats = feat_ref[0]
    ft = feats.T
    pf_ref[...] = ft

    x = pts_ref[0, 0, :][None, :]
    y = pts_ref[0, 1, :][None, :]
    z = pts_ref[0, 2, :][None, :]
    bx = params_ref[0]
    cx = bx[0][:, None]
    cy = bx[1][:, None]
    cz = bx[2][:, None]
    hx = bx[3][:, None]
    hy = bx[4][:, None]
    hz = bx[5][:, None]
    ca = bx[6][:, None]
    sa = bx[7][:, None]

    sx = x - cx
    sy = y - cy
    sz = z - cz
    lx = sx * ca - sy * sa
    ly = sx * sa + sy * ca
    inb = (jnp.abs(lx) <= hx) & (jnp.abs(ly) <= hy) & (jnp.abs(sz) <= hz)
    bi = jax.lax.broadcasted_iota(jnp.int32, (NSEG, NBLK), 0)
    sel = jnp.min(jnp.where(inb, bi, NOBJ), axis=0)

    onehot = sel[None, :] == jax.lax.broadcasted_iota(jnp.int32, (NSEG, NBLK), 0)
    counts = jnp.sum(onehot.astype(jnp.int32), axis=1)

    rows = []
    for j in range(NOBJ):
        m = (sel == j)[:, None]
        rows.append(jnp.max(jnp.where(m, ft, NEG), axis=0))
    blockmax = jnp.concatenate(
        [jnp.stack(rows), jnp.full((NSEG - NOBJ, ft.shape[1]), NEG, ft.dtype)],
        axis=0)

    @pl.when(k == 0)
    def _():
        seg_ref[...] = jnp.full(seg_ref.shape, NEG, seg_ref.dtype)
        cnt_ref[...] = jnp.zeros(cnt_ref.shape, cnt_ref.dtype)

    seg_ref[0] = jnp.maximum(seg_ref[0], blockmax)
    cnt_ref[0, 0] = cnt_ref[0, 0] + counts

    @pl.when(k == nblocks - 1)
    def _():
        alive = (cnt_ref[0, 0] > 0)[:, None]
        seg_ref[0] = jnp.where(alive, seg_ref[0], 0.0)


def kernel(point_features, points, gt_boxes, batch_size):
    bs, c, n_per = point_features.shape
    nobj = gt_boxes.shape[1]
    k_blocks = n_per // NBLK

    pts_t = points[:, 1:4].reshape(bs, n_per, 3).transpose(0, 2, 1)
    gb = jnp.concatenate(
        [gt_boxes[:, :, 0:7],
         jnp.zeros((bs, NSEG - nobj, 7), gt_boxes.dtype)
         .at[:, :, 3:6].set(-1.0)],
        axis=1)
    params = jnp.stack(
        [gb[..., 0], gb[..., 1], gb[..., 2],
         gb[..., 3] * 0.5, gb[..., 4] * 0.5, gb[..., 5] * 0.5,
         jnp.cos(-gb[..., 6]), jnp.sin(-gb[..., 6])], axis=1)

    pf, seg, cnt = pl.pallas_call(
        functools.partial(_tile_kernel, nblocks=k_blocks),
        grid=(bs, k_blocks),
        in_specs=[
            pl.BlockSpec((1, 8, NSEG), lambda b, k: (b, 0, 0)),
            pl.BlockSpec((1, 3, NBLK), lambda b, k: (b, 0, k)),
            pl.BlockSpec((1, c, NBLK), lambda b, k: (b, 0, k)),
        ],
        out_specs=[
            pl.BlockSpec((NBLK, c), lambda b, k: (b * k_blocks + k, 0)),
            pl.BlockSpec((1, NSEG, c), lambda b, k: (b, 0, 0)),
            pl.BlockSpec((1, 1, NSEG), lambda b, k: (b, 0, 0)),
        ],
        out_shape=[
            jax.ShapeDtypeStruct((bs * n_per, c), point_features.dtype),
            jax.ShapeDtypeStruct((bs, NSEG, c), point_features.dtype),
            jax.ShapeDtypeStruct((bs, 1, NSEG), jnp.int32),
        ],
    )(params, pts_t, point_features)

    all_seg = seg[:, :nobj, :].reshape(bs * nobj, c)
    return all_seg, pf


# speedup vs baseline: 1.8553x; 1.8553x over previous
import functools

import jax
import jax.numpy as jnp
from jax.experimental import pallas as pl

NOBJ = 40
NSEG = 48
NBLK = 512
NEG = -3.0e38


def _tile_kernel(params_ref, pts_ref, feat_ref, pf_ref, seg_ref,
                 *, nblocks):
    k = pl.program_id(1)

    feats = feat_ref[0]
    ft = feats.T
    pf_ref[...] = ft

    x = pts_ref[0, :, 1:2]
    y = pts_ref[0, :, 2:3]
    z = pts_ref[0, :, 3:4]
    bx = params_ref[0]
    cx = bx[0:1, :]
    cy = bx[1:2, :]
    cz = bx[2:3, :]
    hx = bx[3:4, :]
    hy = bx[4:5, :]
    hz = bx[5:6, :]
    ca = bx[6:7, :]
    sa = bx[7:8, :]

    sx = x - cx
    sy = y - cy
    sz = z - cz
    lx = sx * ca - sy * sa
    ly = sx * sa + sy * ca
    inb = (jnp.abs(lx) <= hx) & (jnp.abs(ly) <= hy) & (jnp.abs(sz) <= hz)
    bi = jax.lax.broadcasted_iota(jnp.int32, (NBLK, NSEG), 1)
    sel = jnp.min(jnp.where(inb, bi, NOBJ), axis=1, keepdims=True)

    rows = []
    for j in range(NOBJ):
        m = sel == j
        rows.append(jnp.max(jnp.where(m, ft, NEG), axis=0, keepdims=True))
    rows.append(jnp.full((NSEG - NOBJ, ft.shape[1]), NEG, ft.dtype))
    blockmax = jnp.concatenate(rows, axis=0)

    @pl.when(k == 0)
    def _():
        seg_ref[...] = jnp.full(seg_ref.shape, NEG, seg_ref.dtype)

    seg_ref[0] = jnp.maximum(seg_ref[0], blockmax)

    @pl.when(k == nblocks - 1)
    def _():
        cur = seg_ref[0]
        seg_ref[0] = jnp.where(cur < -1.0e38, 0.0, cur)


def kernel(point_features, points, gt_boxes, batch_size):
    bs, c, n_per = point_features.shape
    nobj = gt_boxes.shape[1]
    k_blocks = n_per // NBLK

    pts = points.reshape(bs, n_per, 4)
    gb = jnp.concatenate(
        [gt_boxes[:, :, 0:7],
         jnp.zeros((bs, NSEG - nobj, 7), gt_boxes.dtype)
         .at[:, :, 3:6].set(-1.0)],
        axis=1)
    params = jnp.stack(
        [gb[..., 0], gb[..., 1], gb[..., 2],
         gb[..., 3] * 0.5, gb[..., 4] * 0.5, gb[..., 5] * 0.5,
         jnp.cos(-gb[..., 6]), jnp.sin(-gb[..., 6])], axis=1)

    pf, seg = pl.pallas_call(
        functools.partial(_tile_kernel, nblocks=k_blocks),
        grid=(bs, k_blocks),
        in_specs=[
            pl.BlockSpec((1, 8, NSEG), lambda b, k: (b, 0, 0)),
            pl.BlockSpec((1, NBLK, 4), lambda b, k: (b, k, 0)),
            pl.BlockSpec((1, c, NBLK), lambda b, k: (b, 0, k)),
        ],
        out_specs=[
            pl.BlockSpec((NBLK, c), lambda b, k: (b * k_blocks + k, 0)),
            pl.BlockSpec((1, NSEG, c), lambda b, k: (b, 0, 0)),
        ],
        out_shape=[
            jax.ShapeDtypeStruct((bs * n_per, c), point_features.dtype),
            jax.ShapeDtypeStruct((bs, NSEG, c), point_features.dtype),
        ],
    )(params, pts, point_features)

    all_seg = seg[:, :nobj, :].reshape(bs * nobj, c)
    return all_seg, pf


# speedup vs baseline: 2.1373x; 1.1520x over previous
import functools

import jax
import jax.numpy as jnp
from jax import lax
from jax.experimental import pallas as pl
from jax.experimental.pallas import tpu as pltpu
from jax.experimental.pallas import tpu_sc as plsc

NOBJ = 40
NSEG = 48
NBLK = 512
NEG = -3.0e38
NC = 2
NS = 16
NW = NC * NS
G = 256


def _tile_kernel(params_ref, pts_ref, feat_ref, pf_ref, ids_ref):
    feats = feat_ref[0]
    pf_ref[...] = feats.T

    x = pts_ref[0, 0:1, :]
    y = pts_ref[0, 1:2, :]
    z = pts_ref[0, 2:3, :]
    bx = params_ref[0]
    cx = bx[:, 0:1]
    cy = bx[:, 1:2]
    cz = bx[:, 2:3]
    hx = bx[:, 3:4]
    hy = bx[:, 4:5]
    hz = bx[:, 5:6]
    ca = bx[:, 6:7]
    sa = bx[:, 7:8]

    sx = x - cx
    sy = y - cy
    sz = z - cz
    lx = sx * ca - sy * sa
    ly = sx * sa + sy * ca
    inb = (jnp.abs(lx) <= hx) & (jnp.abs(ly) <= hy) & (jnp.abs(sz) <= hz)
    bi = jax.lax.broadcasted_iota(jnp.int32, (NSEG, NBLK), 0)
    sel = jnp.min(jnp.where(inb, bi, NOBJ), axis=0, keepdims=True)
    ids_ref[...] = sel.reshape(1, 1, NBLK)


def _seg_kernel(pf_hbm, ids_hbm, out_hbm, fbuf, ibuf, acc, *, ppw):
    wid = lax.axis_index("s") * NC + lax.axis_index("c")
    base = wid * ppw
    c = 128

    def init_body(i, _):
        acc[pl.ds(i * 16, 16)] = jnp.full((16,), NEG, jnp.float32)
        return 0
    lax.fori_loop(0, NSEG * c // 16, init_body, 0)

    def chunk_body(g, _):
        row0 = base + g * G
        pltpu.sync_copy(pf_hbm.at[pl.ds(row0, G)], fbuf)
        pltpu.sync_copy(ids_hbm.at[pl.ds(row0, G)], ibuf)

        def group_body(q, _):
            p0 = q * 16
            idsv = ibuf[pl.ds(p0, 16)]

            for j in range(16):
                sid = idsv[j]

                @pl.when(sid < NOBJ)
                def _(sid=sid, j=j):
                    off = sid * c
                    for v in range(8):
                        a = acc[pl.ds(off + v * 16, 16)]
                        f = fbuf[p0 + j, pl.ds(v * 16, 16)]
                        acc[pl.ds(off + v * 16, 16)] = jnp.maximum(a, f)
            return 0
        lax.fori_loop(0, G // 16, group_body, 0)
        return 0
    lax.fori_loop(0, ppw // G, chunk_body, 0)

    pltpu.sync_copy(acc, out_hbm.at[wid])


def _merge_kernel(tab_ref, out_ref, *, nt):
    mx = tab_ref[0, 0]
    for i in range(1, nt):
        mx = jnp.maximum(mx, tab_ref[0, i])
    out_ref[0] = jnp.where(mx < -1.0e38, 0.0, mx)


def kernel(point_features, points, gt_boxes, batch_size):
    bs, c, n_per = point_features.shape
    nobj = gt_boxes.shape[1]
    k_blocks = n_per // NBLK
    ppw = bs * n_per // NW
    tecs_per_scene = NW // bs

    pts_t = points[:, 1:4].reshape(bs, n_per, 3).transpose(0, 2, 1)
    gb = jnp.concatenate(
        [gt_boxes[:, :, 0:7],
         jnp.zeros((bs, NSEG - nobj, 7), gt_boxes.dtype)
         .at[:, :, 3:6].set(-1.0)],
        axis=1)
    params = jnp.stack(
        [gb[..., 0], gb[..., 1], gb[..., 2],
         gb[..., 3] * 0.5, gb[..., 4] * 0.5, gb[..., 5] * 0.5,
         jnp.cos(-gb[..., 6]), jnp.sin(-gb[..., 6])], axis=2)

    pf, ids3 = pl.pallas_call(
        _tile_kernel,
        grid=(bs, k_blocks),
        in_specs=[
            pl.BlockSpec((1, NSEG, 8), lambda b, k: (b, 0, 0)),
            pl.BlockSpec((1, 3, NBLK), lambda b, k: (b, 0, k)),
            pl.BlockSpec((1, c, NBLK), lambda b, k: (b, 0, k)),
        ],
        out_specs=[
            pl.BlockSpec((NBLK, c), lambda b, k: (b * k_blocks + k, 0)),
            pl.BlockSpec((1, 1, NBLK), lambda b, k: (b * k_blocks + k, 0, 0)),
        ],
        out_shape=[
            jax.ShapeDtypeStruct((bs * n_per, c), point_features.dtype),
            jax.ShapeDtypeStruct((bs * k_blocks, 1, NBLK), jnp.int32),
        ],
    )(params, pts_t, point_features)
    ids = ids3.reshape(bs * n_per)

    mesh = plsc.VectorSubcoreMesh(core_axis_name="c", subcore_axis_name="s")
    seg_fn = functools.partial(
        pl.kernel,
        mesh=mesh,
        out_type=jax.ShapeDtypeStruct((NW, NSEG * c), jnp.float32),
        scratch_types=[
            pltpu.VMEM((G, c), jnp.float32),
            pltpu.VMEM((G,), jnp.int32),
            pltpu.VMEM((NSEG * c,), jnp.float32),
        ],
    )(functools.partial(_seg_kernel, ppw=ppw))
    tables = seg_fn(pf, ids)
    tables = tables.reshape(bs, tecs_per_scene, NSEG, c)

    seg = pl.pallas_call(
        functools.partial(_merge_kernel, nt=tecs_per_scene),
        grid=(bs,),
        in_specs=[pl.BlockSpec((1, tecs_per_scene, NSEG, c),
                               lambda b: (b, 0, 0, 0))],
        out_specs=pl.BlockSpec((1, NSEG, c), lambda b: (b, 0, 0)),
        out_shape=jax.ShapeDtypeStruct((bs, NSEG, c), jnp.float32),
    )(tables)

    all_seg = seg[:, :nobj, :].reshape(bs * nobj, c)
    return all_seg, pf
